# baseline (speedup 1.0000x reference)
import functools

import jax
import jax.numpy as jnp
from jax import lax
from jax.experimental import pallas as pl
from jax.experimental.pallas import tpu as pltpu
from jax.experimental.pallas import tpu_sc as plsc

BINS_ = 10
BC = 2048
SC_S = 4096
NC, NS, L = 2, 16, 16
NW = NC * NS
SCHUNK = 128


def _sc_body(x2d, t_hbm, s_hbm, xt_hbm,
             cols_v, lab_v, s_v, xt_v):
    wid = lax.axis_index("s") * NC + lax.axis_index("c")
    per_w = SC_S // NW
    nchunks = per_w // SCHUNK
    base = wid * per_w
    lane = lax.iota(jnp.int32, L)

    def chunk_body(ck, _):
        s0 = base + ck * SCHUNK
        pltpu.sync_copy(x2d.at[:, pl.ds(s0, SCHUNK)], cols_v)
        pltpu.sync_copy(t_hbm.at[pl.ds(s0, SCHUNK)], lab_v)

        for g in range(SCHUNK // L):
            lab16 = lab_v[pl.ds(g * L, L)]
            xt = jnp.zeros((L,), jnp.float32)
            for r in range(L):
                v = cols_v[lab16[r], pl.ds(g * L, L)]
                xt = jnp.where(lane == r, v, xt)
            xt_v[pl.ds(g * L, L)] = xt

        def cbody(c, accs):
            return tuple(
                accs[g] + jnp.exp(cols_v[c, pl.ds(g * L, L)])
                for g in range(SCHUNK // L))

        accs = lax.fori_loop(
            0, 1000, cbody,
            tuple(jnp.zeros((L,), jnp.float32)
                  for _ in range(SCHUNK // L)))
        for g in range(SCHUNK // L):
            s_v[pl.ds(g * L, L)] = accs[g]
        pltpu.sync_copy(s_v, s_hbm.at[pl.ds(s0, SCHUNK)])
        pltpu.sync_copy(xt_v, xt_hbm.at[pl.ds(s0, SCHUNK)])
        return 0

    lax.fori_loop(0, nchunks, chunk_body, 0)


def _part(x, labels):
    s = jnp.sum(jnp.exp(x), axis=0, keepdims=True)
    rows = jax.lax.broadcasted_iota(jnp.int32, x.shape, 0)
    xt = jnp.sum(jnp.where(rows == labels, x, 0.0), axis=0, keepdims=True)
    p = jnp.exp(xt) / s
    bin_raw = jnp.floor((1.0 - p) * BINS_).astype(jnp.int32)
    sel = (bin_raw >= 0) & (bin_raw < BINS_)
    logp = xt - jnp.log(s)
    return bin_raw, sel, logp


def _tc_kernel(x0_ref, x1_ref, t0_ref, t1_ref, out_ref, acc_ref, *, nsteps):
    i = pl.program_id(0)

    @pl.when(i == 0)
    def _init():
        acc_ref[...] = jnp.zeros_like(acc_ref)

    b0, s0, l0 = _part(x0_ref[...], t0_ref[...])
    b1, s1, l1 = _part(x1_ref[...], t1_ref[...])
    cnts = []
    slogs = []
    for b in range(BINS_):
        m0 = (b0 == b) & s0
        m1 = (b1 == b) & s1
        c = (jnp.sum(m0.astype(jnp.float32), keepdims=True)
             + jnp.sum(m1.astype(jnp.float32), keepdims=True))
        sl = (jnp.sum(jnp.where(m0, l0, 0.0), keepdims=True)
              + jnp.sum(jnp.where(m1, l1, 0.0), keepdims=True))
        cnts.append(c.reshape(1, 1))
        slogs.append(sl.reshape(1, 1))
    acc_ref[0:1, :] += jnp.concatenate(cnts, axis=1)
    acc_ref[1:2, :] += jnp.concatenate(slogs, axis=1)

    @pl.when(i == nsteps - 1)
    def _fin():
        out_ref[...] = acc_ref[...]


def _finalize(s_ref, xt_ref, acc_ref, out_ref):
    s = s_ref[...]
    xt = xt_ref[...]
    p = jnp.exp(xt) / s
    logp = xt - jnp.log(s)
    bin_raw = jnp.floor((1.0 - p) * BINS_).astype(jnp.int32)
    sel = (bin_raw >= 0) & (bin_raw < BINS_)
    cnts = []
    slogs = []
    for b in range(BINS_):
        m = (bin_raw == b) & sel
        cnts.append(jnp.sum(m.astype(jnp.float32), keepdims=True)
                    .reshape(1, 1))
        slogs.append(jnp.sum(jnp.where(m, logp, 0.0), keepdims=True)
                     .reshape(1, 1))
    counts = jnp.concatenate(cnts, axis=1) + acc_ref[0:1, :]
    slog = jnp.concatenate(slogs, axis=1) + acc_ref[1:2, :]
    nonempty = counts > 0
    n = jnp.sum(nonempty.astype(jnp.float32), keepdims=True)
    per_bin = jnp.where(nonempty, slog / jnp.maximum(counts, 1.0), 0.0)
    out_ref[...] = -jnp.sum(per_bin, keepdims=True) / jnp.maximum(n, 1.0)


def kernel(y_pred, y_true):
    n, c = y_pred.shape
    xT = y_pred.T
    tl = y_true.reshape(1, n)

    sc_kernel = pl.kernel(
        _sc_body,
        out_type=[
            jax.ShapeDtypeStruct((SC_S,), jnp.float32),
            jax.ShapeDtypeStruct((SC_S,), jnp.float32),
        ],
        mesh=plsc.VectorSubcoreMesh(core_axis_name="c", subcore_axis_name="s"),
        scratch_types=[
            pltpu.VMEM((c, SCHUNK), jnp.float32),
            pltpu.VMEM((SCHUNK,), jnp.int32),
            pltpu.VMEM((SCHUNK,), jnp.float32),
            pltpu.VMEM((SCHUNK,), jnp.float32),
        ],
    )
    s_sc, xt_sc = sc_kernel(xT, y_true)

    tc_n = n - SC_S
    nsteps = tc_n // (BC * 2)
    off = SC_S // BC
    acc = pl.pallas_call(
        functools.partial(_tc_kernel, nsteps=nsteps),
        grid=(nsteps,),
        in_specs=[
            pl.BlockSpec((c, BC), lambda i: (0, i + off)),
            pl.BlockSpec((c, BC), lambda i: (0, i + off + nsteps)),
            pl.BlockSpec((1, BC), lambda i: (0, i + off)),
            pl.BlockSpec((1, BC), lambda i: (0, i + off + nsteps)),
        ],
        out_specs=pl.BlockSpec((2, BINS_), lambda i: (0, 0)),
        out_shape=jax.ShapeDtypeStruct((2, BINS_), jnp.float32),
        scratch_shapes=[pltpu.VMEM((2, BINS_), jnp.float32)],
    )(xT, xT, tl, tl)

    s2 = s_sc.reshape(SC_S // 128, 128)
    xt2 = xt_sc.reshape(SC_S // 128, 128)
    out = pl.pallas_call(
        _finalize,
        in_specs=[
            pl.BlockSpec(s2.shape, lambda: (0, 0)),
            pl.BlockSpec(xt2.shape, lambda: (0, 0)),
            pl.BlockSpec((2, BINS_), lambda: (0, 0)),
        ],
        out_specs=pl.BlockSpec((1, 1), lambda: (0, 0)),
        out_shape=jax.ShapeDtypeStruct((1, 1), jnp.float32),
    )(s2, xt2, acc)
    return out[0, 0]


# speedup vs baseline: 1.2320x; 1.2320x over previous
import functools

import jax
import jax.numpy as jnp
from jax.experimental import pallas as pl
from jax.experimental.pallas import tpu as pltpu

BINS_ = 10
BC = 1024


def _part(x, labels):
    s = jnp.sum(jnp.exp(x), axis=0, keepdims=True)
    rows = jax.lax.broadcasted_iota(jnp.int32, x.shape, 0)
    xt = jnp.sum(jnp.where(rows == labels, x, 0.0), axis=0, keepdims=True)
    p = jnp.exp(xt) / s
    bin_raw = jnp.floor((1.0 - p) * BINS_).astype(jnp.int32)
    sel = (bin_raw >= 0) & (bin_raw < BINS_)
    logp = xt - jnp.log(s)
    return bin_raw, sel, logp


def _t_kernel(x0_ref, x1_ref, x2_ref, x3_ref, t0_ref, t1_ref, t2_ref, t3_ref,
              out_ref, acc_ref, *, nsteps):
    i = pl.program_id(0)

    @pl.when(i == 0)
    def _init():
        acc_ref[...] = jnp.zeros_like(acc_ref)

    parts = [_part(x0_ref[...], t0_ref[...]), _part(x1_ref[...], t1_ref[...]),
             _part(x2_ref[...], t2_ref[...]), _part(x3_ref[...], t3_ref[...])]
    cnts = []
    slogs = []
    for b in range(BINS_):
        ms = [(bb == b) & ss for bb, ss, _ in parts]
        c = sum(jnp.sum(m.astype(jnp.float32), keepdims=True) for m in ms)
        sl = sum(jnp.sum(jnp.where(m, ll, 0.0), keepdims=True)
                 for m, (_, _, ll) in zip(ms, parts))
        cnts.append(c.reshape(1, 1))
        slogs.append(sl.reshape(1, 1))
    acc_ref[0:1, :] += jnp.concatenate(cnts, axis=1)
    acc_ref[1:2, :] += jnp.concatenate(slogs, axis=1)

    @pl.when(i == nsteps - 1)
    def _fin():
        counts = acc_ref[0:1, :]
        slog = acc_ref[1:2, :]
        nonempty = counts > 0
        n = jnp.sum(nonempty.astype(jnp.float32), keepdims=True)
        per_bin = jnp.where(nonempty, slog / jnp.maximum(counts, 1.0), 0.0)
        out_ref[...] = (-jnp.sum(per_bin, keepdims=True)
                        / jnp.maximum(n, 1.0))


def kernel(y_pred, y_true):
    n, c = y_pred.shape
    xT = y_pred.T
    tl = y_true.reshape(1, n)
    nsteps = n // (BC * 4)
    out = pl.pallas_call(
        functools.partial(_t_kernel, nsteps=nsteps),
        grid=(nsteps,),
        in_specs=[
            pl.BlockSpec((c, BC), lambda i: (0, i)),
            pl.BlockSpec((c, BC), lambda i: (0, i + nsteps)),
            pl.BlockSpec((c, BC), lambda i: (0, i + 2 * nsteps)),
            pl.BlockSpec((c, BC), lambda i: (0, i + 3 * nsteps)),
            pl.BlockSpec((1, BC), lambda i: (0, i)),
            pl.BlockSpec((1, BC), lambda i: (0, i + nsteps)),
            pl.BlockSpec((1, BC), lambda i: (0, i + 2 * nsteps)),
            pl.BlockSpec((1, BC), lambda i: (0, i + 3 * nsteps)),
        ],
        out_specs=pl.BlockSpec((1, 1), lambda i: (0, 0)),
        out_shape=jax.ShapeDtypeStruct((1, 1), jnp.float32),
        scratch_shapes=[pltpu.VMEM((2, BINS_), jnp.float32)],
    )(xT, xT, xT, xT, tl, tl, tl, tl)
    return out[0, 0]


# speedup vs baseline: 1.2721x; 1.0325x over previous
import functools

import jax
import jax.numpy as jnp
from jax.experimental import pallas as pl
from jax.experimental.pallas import tpu as pltpu

BINS_ = 10
BC = 512


def _part(x, labels):
    s = jnp.sum(jnp.exp(x), axis=0, keepdims=True)
    rows = jax.lax.broadcasted_iota(jnp.int32, x.shape, 0)
    xt = jnp.sum(jnp.where(rows == labels, x, 0.0), axis=0, keepdims=True)
    p = jnp.exp(xt) / s
    bin_raw = jnp.floor((1.0 - p) * BINS_).astype(jnp.int32)
    sel = (bin_raw >= 0) & (bin_raw < BINS_)
    logp = xt - jnp.log(s)
    return bin_raw, sel, logp


def _t_kernel(x0_ref, x1_ref, x2_ref, x3_ref, x4_ref, x5_ref, x6_ref, x7_ref,
              t0_ref, t1_ref, t2_ref, t3_ref, t4_ref, t5_ref, t6_ref, t7_ref,
              out_ref, acc_ref, *, nsteps):
    i = pl.program_id(0)

    @pl.when(i == 0)
    def _init():
        acc_ref[...] = jnp.zeros_like(acc_ref)

    parts = [_part(x0_ref[...], t0_ref[...]), _part(x1_ref[...], t1_ref[...]),
             _part(x2_ref[...], t2_ref[...]), _part(x3_ref[...], t3_ref[...]),
             _part(x4_ref[...], t4_ref[...]), _part(x5_ref[...], t5_ref[...]),
             _part(x6_ref[...], t6_ref[...]), _part(x7_ref[...], t7_ref[...])]
    cnts = []
    slogs = []
    for b in range(BINS_):
        ms = [(bb == b) & ss for bb, ss, _ in parts]
        c = sum(jnp.sum(m.astype(jnp.float32), keepdims=True) for m in ms)
        sl = sum(jnp.sum(jnp.where(m, ll, 0.0), keepdims=True)
                 for m, (_, _, ll) in zip(ms, parts))
        cnts.append(c.reshape(1, 1))
        slogs.append(sl.reshape(1, 1))
    acc_ref[0:1, :] += jnp.concatenate(cnts, axis=1)
    acc_ref[1:2, :] += jnp.concatenate(slogs, axis=1)

    @pl.when(i == nsteps - 1)
    def _fin():
        counts = acc_ref[0:1, :]
        slog = acc_ref[1:2, :]
        nonempty = counts > 0
        n = jnp.sum(nonempty.astype(jnp.float32), keepdims=True)
        per_bin = jnp.where(nonempty, slog / jnp.maximum(counts, 1.0), 0.0)
        out_ref[...] = (-jnp.sum(per_bin, keepdims=True)
                        / jnp.maximum(n, 1.0))


def kernel(y_pred, y_true):
    n, c = y_pred.shape
    xT = y_pred.T
    tl = y_true.reshape(1, n)
    nsteps = n // (BC * 8)
    out = pl.pallas_call(
        functools.partial(_t_kernel, nsteps=nsteps),
        grid=(nsteps,),
        in_specs=(
            [pl.BlockSpec((c, BC), functools.partial(
                lambda k, i: (0, i + k * nsteps), k)) for k in range(8)]
            + [pl.BlockSpec((1, BC), functools.partial(
                lambda k, i: (0, i + k * nsteps), k)) for k in range(8)]),
        out_specs=pl.BlockSpec((1, 1), lambda i: (0, 0)),
        out_shape=jax.ShapeDtypeStruct((1, 1), jnp.float32),
        scratch_shapes=[pltpu.VMEM((2, BINS_), jnp.float32)],
    )(*([xT] * 8), *([tl] * 8))
    return out[0, 0]
